# Initial kernel scaffold; baseline (speedup 1.0000x reference)
import jax
import jax.numpy as jnp
from jax.experimental import pallas as pl


def kernel(input, embed):
    raise NotImplementedError("write your pallas kernel here")


# speedup vs baseline: 1.1323x; 1.1323x over previous
import functools

import jax
import jax.numpy as jnp
from jax import lax
from jax.experimental import pallas as pl
from jax.experimental.pallas import tpu as pltpu
from jax.experimental.pallas import tpu_sc as plsc

_DIM = 32
_K = 8192
_BN = 256


def _chunk_argmax(neg):
    m = jnp.max(neg, axis=1, keepdims=True)
    iota = jax.lax.broadcasted_iota(jnp.int32, neg.shape, 1)
    idx = jnp.min(jnp.where(neg == m, iota, _K), axis=1).astype(jnp.int32)
    return m[:, 0], idx


def _dist_argmin_body(x_ref, e_ref, e2_ref, ind_ref, dsum_ref):
    x = x_ref[...]
    e = e_ref[...]
    e2 = e2_ref[...]
    x2 = jnp.sum(x * x, axis=1, keepdims=True)
    neg = -(x2 - 2.0 * jnp.dot(x, e) + e2)
    half = _K // 2
    m0, i0 = _chunk_argmax(neg[:, :half])
    m1, i1 = _chunk_argmax(neg[:, half:])
    m0r = m0.astype(jnp.bfloat16).astype(jnp.float32)
    win1 = m1 > m0r
    ind_ref[...] = jnp.where(win1, i1 + half, i0)
    mind = -jnp.where(win1, m1, m0)

    @pl.when(pl.program_id(0) == 0)
    def _():
        dsum_ref[...] = jnp.zeros_like(dsum_ref)

    dsum_ref[...] += jnp.sum(mind).reshape(1, 1)


def _argmin_call(flatten, embed, e2):
    n = flatten.shape[0]
    grid = (n // _BN,)
    return pl.pallas_call(
        _dist_argmin_body,
        grid=grid,
        in_specs=[
            pl.BlockSpec((_BN, _DIM), lambda i: (i, 0)),
            pl.BlockSpec((_DIM, _K), lambda i: (0, 0)),
            pl.BlockSpec((1, _K), lambda i: (0, 0)),
        ],
        out_specs=[
            pl.BlockSpec((_BN,), lambda i: (i,)),
            pl.BlockSpec((1, 1), lambda i: (0, 0)),
        ],
        out_shape=[
            jax.ShapeDtypeStruct((n,), jnp.int32),
            jax.ShapeDtypeStruct((1, 1), jnp.float32),
        ],
        compiler_params=pltpu.CompilerParams(
            dimension_semantics=("arbitrary",),
        ),
    )(flatten, embed, e2)


@functools.lru_cache(maxsize=None)
def _make_gather(n_rows):
    info = plsc.get_sparse_core_info()
    nc, ns = info.num_cores, info.num_subcores
    nw = nc * ns
    b_per_w = n_rows // nw
    chunk = 128
    n_chunks = b_per_w // chunk
    mesh = plsc.VectorSubcoreMesh(core_axis_name="c", subcore_axis_name="s")

    @functools.partial(
        pl.kernel,
        mesh=mesh,
        out_type=jax.ShapeDtypeStruct((n_rows, _DIM), jnp.float32),
        scratch_types=[
            pltpu.VMEM((n_chunks, chunk), jnp.int32),
            pltpu.VMEM((b_per_w, _DIM), jnp.float32),
            pltpu.SemaphoreType.DMA,
        ],
        compiler_params=pltpu.CompilerParams(use_tc_tiling_on_sc=False),
    )
    def gather(table_hbm, idx_hbm, out_hbm, idx_v, rows_v, sem):
        wid = lax.axis_index("s") * nc + lax.axis_index("c")
        base = wid * b_per_w
        pltpu.sync_copy(idx_hbm.at[pl.ds(wid * n_chunks, n_chunks)], idx_v)
        copies = [
            pltpu.make_async_copy(
                table_hbm.at[idx_v.at[j]],
                rows_v.at[pl.ds(j * chunk, chunk)],
                sem,
            )
            for j in range(n_chunks)
        ]
        for c in copies:
            c.start()
        for c in copies:
            c.wait()
        pltpu.sync_copy(rows_v, out_hbm.at[pl.ds(base, b_per_w)])

    return gather


def kernel(input, embed):
    shape = input.shape
    n = shape[0] * shape[1] * shape[2]
    flatten = input.reshape(-1, _DIM)
    e2 = jnp.sum(embed ** 2, axis=0, keepdims=True)
    ind, dsum = _argmin_call(flatten, embed, e2)
    table = embed.T
    idx2d = ind.reshape(n // 128, 128)
    q = _make_gather(n)(table, idx2d)
    quantize = q.reshape(shape)
    diff = dsum[0, 0] / jnp.float32(n * _DIM)
    quantize = input + lax.stop_gradient(quantize - input)
    embed_ind = ind.reshape(shape[:-1])
    return quantize, diff, embed_ind


# speedup vs baseline: 1.3596x; 1.2007x over previous
import functools

import jax
import jax.numpy as jnp
from jax import lax
from jax.experimental import pallas as pl
from jax.experimental.pallas import tpu as pltpu
from jax.experimental.pallas import tpu_sc as plsc

_DIM = 32
_K = 8192
_BN = 256


def _chunk_argmin(d):
    m = jnp.min(d, axis=1, keepdims=True)
    iota = jax.lax.broadcasted_iota(jnp.int32, d.shape, 1)
    idx = jnp.min(jnp.where(d == m, iota, _K), axis=1).astype(jnp.int32)
    return m[:, 0], idx


def _dist_argmin_body(x_ref, e_ref, e2_ref, ind_ref, dsum_ref):
    x = x_ref[...]
    e2x = e_ref[...]
    e2 = e2_ref[...]
    x2 = jnp.sum(x * x, axis=1, keepdims=True)
    d = (x2 - jnp.dot(x, e2x)) + e2
    half = _K // 2
    m0, i0 = _chunk_argmin(d[:, :half])
    m1, i1 = _chunk_argmin(d[:, half:])
    m0r = m0.astype(jnp.bfloat16).astype(jnp.float32)
    win1 = m1 < m0r
    ind_ref[...] = jnp.where(win1, i1 + half, i0)
    mind = jnp.where(win1, m1, m0)

    @pl.when(pl.program_id(0) == 0)
    def _():
        dsum_ref[...] = jnp.zeros_like(dsum_ref)

    dsum_ref[...] += jnp.sum(mind).reshape(1, 1)


def _argmin_call(flatten, embed, e2):
    n = flatten.shape[0]
    grid = (n // _BN,)
    return pl.pallas_call(
        _dist_argmin_body,
        grid=grid,
        in_specs=[
            pl.BlockSpec((_BN, _DIM), lambda i: (i, 0)),
            pl.BlockSpec((_DIM, _K), lambda i: (0, 0)),
            pl.BlockSpec((1, _K), lambda i: (0, 0)),
        ],
        out_specs=[
            pl.BlockSpec((_BN,), lambda i: (i,)),
            pl.BlockSpec((1, 1), lambda i: (0, 0)),
        ],
        out_shape=[
            jax.ShapeDtypeStruct((n,), jnp.int32),
            jax.ShapeDtypeStruct((1, 1), jnp.float32),
        ],
        compiler_params=pltpu.CompilerParams(
            dimension_semantics=("arbitrary",),
        ),
    )(flatten, embed, e2)


@functools.lru_cache(maxsize=None)
def _make_gather(n_rows):
    info = plsc.get_sparse_core_info()
    nc, ns = info.num_cores, info.num_subcores
    nw = nc * ns
    b_per_w = n_rows // nw
    chunk = 128
    n_chunks = b_per_w // chunk
    mesh = plsc.VectorSubcoreMesh(core_axis_name="c", subcore_axis_name="s")

    @functools.partial(
        pl.kernel,
        mesh=mesh,
        out_type=jax.ShapeDtypeStruct((n_rows, _DIM), jnp.float32),
        scratch_types=[
            pltpu.VMEM((n_chunks, chunk), jnp.int32),
            pltpu.VMEM((b_per_w, _DIM), jnp.float32),
            pltpu.SemaphoreType.DMA,
        ],
        compiler_params=pltpu.CompilerParams(use_tc_tiling_on_sc=False),
    )
    def gather(table_hbm, idx_hbm, out_hbm, idx_v, rows_v, sem):
        wid = lax.axis_index("s") * nc + lax.axis_index("c")
        base = wid * b_per_w
        pltpu.sync_copy(idx_hbm.at[pl.ds(wid * n_chunks, n_chunks)], idx_v)
        copies = [
            pltpu.make_async_copy(
                table_hbm.at[idx_v.at[j]],
                rows_v.at[pl.ds(j * chunk, chunk)],
                sem,
            )
            for j in range(n_chunks)
        ]
        for c in copies:
            c.start()
        for c in copies:
            c.wait()
        pltpu.sync_copy(rows_v, out_hbm.at[pl.ds(base, b_per_w)])

    return gather


def kernel(input, embed):
    shape = input.shape
    n = shape[0] * shape[1] * shape[2]
    flatten = input.reshape(-1, _DIM)
    e2 = jnp.sum(embed ** 2, axis=0, keepdims=True)
    ind, dsum = _argmin_call(flatten, embed + embed, e2)
    table = embed.T
    idx2d = ind.reshape(n // 128, 128)
    q = _make_gather(n)(table, idx2d)
    quantize = q.reshape(shape)
    diff = dsum[0, 0] / jnp.float32(n * _DIM)
    quantize = input + lax.stop_gradient(quantize - input)
    embed_ind = ind.reshape(shape[:-1])
    return quantize, diff, embed_ind


# speedup vs baseline: 1.6400x; 1.2063x over previous
import functools

import jax
import jax.numpy as jnp
from jax import lax
from jax.experimental import pallas as pl
from jax.experimental.pallas import tpu as pltpu
from jax.experimental.pallas import tpu_sc as plsc

_DIM = 32
_K = 8192
_BN = 256


_C = 128


def _chunk_argmin(dot, x2, e2, base, half):
    acc_v = (x2 - dot[:, base:base + _C]) + e2[:, base:base + _C]
    acc_j = jnp.zeros(acc_v.shape, jnp.int32)
    for c in range(1, half // _C):
        sl = slice(base + c * _C, base + (c + 1) * _C)
        dc = (x2 - dot[:, sl]) + e2[:, sl]
        le = acc_v <= dc
        acc_j = jnp.where(le, acc_j, c * _C)
        acc_v = jnp.minimum(acc_v, dc)
    m = jnp.min(acc_v, axis=1)
    lane = jax.lax.broadcasted_iota(jnp.int32, acc_v.shape, 1)
    cand = jnp.where(acc_v == m[:, None], acc_j + lane, _K)
    idx = jnp.min(cand, axis=1)
    return m, idx


def _dist_argmin_body(x_ref, e_ref, e2_ref, ind_ref, dsum_ref):
    x = x_ref[...]
    e2 = e2_ref[...]
    x2 = jnp.sum(x * x, axis=1, keepdims=True)
    dot = jnp.dot(x, e_ref[...])
    half = _K // 2
    m0, i0 = _chunk_argmin(dot, x2, e2, 0, half)
    m1, i1 = _chunk_argmin(dot, x2, e2, half, half)
    m0r = m0.astype(jnp.bfloat16).astype(jnp.float32)
    win1 = m1 < m0r
    ind_ref[...] = jnp.where(win1, i1 + half, i0)
    mind = jnp.where(win1, m1, m0)

    @pl.when(pl.program_id(0) == 0)
    def _():
        dsum_ref[...] = jnp.zeros_like(dsum_ref)

    dsum_ref[...] += jnp.sum(mind).reshape(1, 1)


def _argmin_call(flatten, embed, e2):
    n = flatten.shape[0]
    grid = (n // _BN,)
    return pl.pallas_call(
        _dist_argmin_body,
        grid=grid,
        in_specs=[
            pl.BlockSpec((_BN, _DIM), lambda i: (i, 0)),
            pl.BlockSpec((_DIM, _K), lambda i: (0, 0)),
            pl.BlockSpec((1, _K), lambda i: (0, 0)),
        ],
        out_specs=[
            pl.BlockSpec((_BN,), lambda i: (i,)),
            pl.BlockSpec((1, 1), lambda i: (0, 0)),
        ],
        out_shape=[
            jax.ShapeDtypeStruct((n,), jnp.int32),
            jax.ShapeDtypeStruct((1, 1), jnp.float32),
        ],
        compiler_params=pltpu.CompilerParams(
            dimension_semantics=("arbitrary",),
        ),
    )(flatten, embed, e2)


@functools.lru_cache(maxsize=None)
def _make_gather(n_rows):
    info = plsc.get_sparse_core_info()
    nc, ns = info.num_cores, info.num_subcores
    nw = nc * ns
    b_per_w = n_rows // nw
    chunk = 128
    n_chunks = b_per_w // chunk
    mesh = plsc.VectorSubcoreMesh(core_axis_name="c", subcore_axis_name="s")

    @functools.partial(
        pl.kernel,
        mesh=mesh,
        out_type=jax.ShapeDtypeStruct((n_rows, _DIM), jnp.float32),
        scratch_types=[
            pltpu.VMEM((n_chunks, chunk), jnp.int32),
            pltpu.VMEM((b_per_w, _DIM), jnp.float32),
            pltpu.SemaphoreType.DMA,
        ],
        compiler_params=pltpu.CompilerParams(use_tc_tiling_on_sc=False),
    )
    def gather(table_hbm, idx_hbm, out_hbm, idx_v, rows_v, sem):
        wid = lax.axis_index("s") * nc + lax.axis_index("c")
        base = wid * b_per_w
        pltpu.sync_copy(idx_hbm.at[pl.ds(wid * n_chunks, n_chunks)], idx_v)
        copies = [
            pltpu.make_async_copy(
                table_hbm.at[idx_v.at[j]],
                rows_v.at[pl.ds(j * chunk, chunk)],
                sem,
            )
            for j in range(n_chunks)
        ]
        for c in copies:
            c.start()
        for c in copies:
            c.wait()
        pltpu.sync_copy(rows_v, out_hbm.at[pl.ds(base, b_per_w)])

    return gather


def kernel(input, embed):
    shape = input.shape
    n = shape[0] * shape[1] * shape[2]
    flatten = input.reshape(-1, _DIM)
    e2 = jnp.sum(embed ** 2, axis=0, keepdims=True)
    ind, dsum = _argmin_call(flatten, embed + embed, e2)
    table = embed.T
    idx2d = ind.reshape(n // 128, 128)
    q = _make_gather(n)(table, idx2d)
    quantize = q.reshape(shape)
    diff = dsum[0, 0] / jnp.float32(n * _DIM)
    quantize = input + lax.stop_gradient(quantize - input)
    embed_ind = ind.reshape(shape[:-1])
    return quantize, diff, embed_ind


# speedup vs baseline: 1.7304x; 1.0551x over previous
import functools

import jax
import jax.numpy as jnp
from jax import lax
from jax.experimental import pallas as pl
from jax.experimental.pallas import tpu as pltpu
from jax.experimental.pallas import tpu_sc as plsc

_DIM = 32
_K = 8192
_BN = 512


_C = 128


def _chunk_argmin(dot, x2, e2, base, half):
    acc_v = (x2 - dot[:, base:base + _C]) + e2[:, base:base + _C]
    acc_j = jnp.zeros(acc_v.shape, jnp.int32)
    for c in range(1, half // _C):
        sl = slice(base + c * _C, base + (c + 1) * _C)
        dc = (x2 - dot[:, sl]) + e2[:, sl]
        le = acc_v <= dc
        acc_j = jnp.where(le, acc_j, c * _C)
        acc_v = jnp.minimum(acc_v, dc)
    m = jnp.min(acc_v, axis=1)
    lane = jax.lax.broadcasted_iota(jnp.int32, acc_v.shape, 1)
    cand = jnp.where(acc_v == m[:, None], acc_j + lane, _K)
    idx = jnp.min(cand, axis=1)
    return m, idx


def _dist_argmin_body(x_ref, e_ref, e2_ref, ind_ref, dsum_ref):
    x = x_ref[...]
    e2 = e2_ref[...]
    x2 = jnp.sum(x * x, axis=1, keepdims=True)
    dot = jnp.dot(x, e_ref[...])
    half = _K // 2
    m0, i0 = _chunk_argmin(dot, x2, e2, 0, half)
    m1, i1 = _chunk_argmin(dot, x2, e2, half, half)
    m0r = m0.astype(jnp.bfloat16).astype(jnp.float32)
    win1 = m1 < m0r
    ind_ref[...] = jnp.where(win1, i1 + half, i0)
    mind = jnp.where(win1, m1, m0)

    @pl.when(pl.program_id(0) == 0)
    def _():
        dsum_ref[...] = jnp.zeros_like(dsum_ref)

    dsum_ref[...] += jnp.sum(mind).reshape(1, 1)


def _argmin_call(flatten, embed, e2):
    n = flatten.shape[0]
    grid = (n // _BN,)
    return pl.pallas_call(
        _dist_argmin_body,
        grid=grid,
        in_specs=[
            pl.BlockSpec((_BN, _DIM), lambda i: (i, 0)),
            pl.BlockSpec((_DIM, _K), lambda i: (0, 0)),
            pl.BlockSpec((1, _K), lambda i: (0, 0)),
        ],
        out_specs=[
            pl.BlockSpec((_BN,), lambda i: (i,)),
            pl.BlockSpec((1, 1), lambda i: (0, 0)),
        ],
        out_shape=[
            jax.ShapeDtypeStruct((n,), jnp.int32),
            jax.ShapeDtypeStruct((1, 1), jnp.float32),
        ],
        compiler_params=pltpu.CompilerParams(
            dimension_semantics=("arbitrary",),
        ),
    )(flatten, embed, e2)


@functools.lru_cache(maxsize=None)
def _make_gather(n_rows):
    info = plsc.get_sparse_core_info()
    nc, ns = info.num_cores, info.num_subcores
    nw = nc * ns
    b_per_w = n_rows // nw
    chunk = 128
    n_chunks = b_per_w // chunk
    mesh = plsc.VectorSubcoreMesh(core_axis_name="c", subcore_axis_name="s")

    @functools.partial(
        pl.kernel,
        mesh=mesh,
        out_type=jax.ShapeDtypeStruct((n_rows, _DIM), jnp.float32),
        scratch_types=[
            pltpu.VMEM((n_chunks, chunk), jnp.int32),
            pltpu.VMEM((b_per_w, _DIM), jnp.float32),
            pltpu.SemaphoreType.DMA,
        ],
        compiler_params=pltpu.CompilerParams(use_tc_tiling_on_sc=False),
    )
    def gather(table_hbm, idx_hbm, out_hbm, idx_v, rows_v, sem):
        wid = lax.axis_index("s") * nc + lax.axis_index("c")
        base = wid * b_per_w
        pltpu.sync_copy(idx_hbm.at[pl.ds(wid * n_chunks, n_chunks)], idx_v)
        copies = [
            pltpu.make_async_copy(
                table_hbm.at[idx_v.at[j]],
                rows_v.at[pl.ds(j * chunk, chunk)],
                sem,
            )
            for j in range(n_chunks)
        ]
        for c in copies:
            c.start()
        for c in copies:
            c.wait()
        pltpu.sync_copy(rows_v, out_hbm.at[pl.ds(base, b_per_w)])

    return gather


def kernel(input, embed):
    shape = input.shape
    n = shape[0] * shape[1] * shape[2]
    flatten = input.reshape(-1, _DIM)
    e2 = jnp.sum(embed ** 2, axis=0, keepdims=True)
    ind, dsum = _argmin_call(flatten, embed + embed, e2)
    table = embed.T
    idx2d = ind.reshape(n // 128, 128)
    q = _make_gather(n)(table, idx2d)
    quantize = q.reshape(shape)
    diff = dsum[0, 0] / jnp.float32(n * _DIM)
    quantize = input + lax.stop_gradient(quantize - input)
    embed_ind = ind.reshape(shape[:-1])
    return quantize, diff, embed_ind


# speedup vs baseline: 1.7716x; 1.0238x over previous
import functools

import jax
import jax.numpy as jnp
from jax import lax
from jax.experimental import pallas as pl
from jax.experimental.pallas import tpu as pltpu
from jax.experimental.pallas import tpu_sc as plsc

_DIM = 32
_K = 8192
_BN = 1024


_C = 128


def _chunk_argmin(dot, x2, e2, base, half):
    acc_v = (x2 - dot[:, base:base + _C]) + e2[:, base:base + _C]
    acc_j = jnp.zeros(acc_v.shape, jnp.int32)
    for c in range(1, half // _C):
        sl = slice(base + c * _C, base + (c + 1) * _C)
        dc = (x2 - dot[:, sl]) + e2[:, sl]
        le = acc_v <= dc
        acc_j = jnp.where(le, acc_j, c * _C)
        acc_v = jnp.minimum(acc_v, dc)
    m = jnp.min(acc_v, axis=1)
    lane = jax.lax.broadcasted_iota(jnp.int32, acc_v.shape, 1)
    cand = jnp.where(acc_v == m[:, None], acc_j + lane, _K)
    idx = jnp.min(cand, axis=1)
    return m, idx


def _dist_argmin_body(x_ref, e_ref, e2_ref, ind_ref, dsum_ref):
    x = x_ref[...]
    e2 = e2_ref[...]
    x2 = jnp.sum(x * x, axis=1, keepdims=True)
    dot = jnp.dot(x, e_ref[...])
    half = _K // 2
    m0, i0 = _chunk_argmin(dot, x2, e2, 0, half)
    m1, i1 = _chunk_argmin(dot, x2, e2, half, half)
    m0r = m0.astype(jnp.bfloat16).astype(jnp.float32)
    win1 = m1 < m0r
    ind_ref[...] = jnp.where(win1, i1 + half, i0)
    mind = jnp.where(win1, m1, m0)

    @pl.when(pl.program_id(0) == 0)
    def _():
        dsum_ref[...] = jnp.zeros_like(dsum_ref)

    dsum_ref[...] += jnp.sum(mind).reshape(1, 1)


def _argmin_call(flatten, embed, e2):
    n = flatten.shape[0]
    grid = (n // _BN,)
    return pl.pallas_call(
        _dist_argmin_body,
        grid=grid,
        in_specs=[
            pl.BlockSpec((_BN, _DIM), lambda i: (i, 0)),
            pl.BlockSpec((_DIM, _K), lambda i: (0, 0)),
            pl.BlockSpec((1, _K), lambda i: (0, 0)),
        ],
        out_specs=[
            pl.BlockSpec((_BN,), lambda i: (i,)),
            pl.BlockSpec((1, 1), lambda i: (0, 0)),
        ],
        out_shape=[
            jax.ShapeDtypeStruct((n,), jnp.int32),
            jax.ShapeDtypeStruct((1, 1), jnp.float32),
        ],
        compiler_params=pltpu.CompilerParams(
            dimension_semantics=("arbitrary",),
        ),
    )(flatten, embed, e2)


@functools.lru_cache(maxsize=None)
def _make_gather(n_rows):
    info = plsc.get_sparse_core_info()
    nc, ns = info.num_cores, info.num_subcores
    nw = nc * ns
    b_per_w = n_rows // nw
    chunk = 128
    n_chunks = b_per_w // chunk
    mesh = plsc.VectorSubcoreMesh(core_axis_name="c", subcore_axis_name="s")

    @functools.partial(
        pl.kernel,
        mesh=mesh,
        out_type=jax.ShapeDtypeStruct((n_rows, _DIM), jnp.float32),
        scratch_types=[
            pltpu.VMEM((n_chunks, chunk), jnp.int32),
            pltpu.VMEM((b_per_w, _DIM), jnp.float32),
            pltpu.SemaphoreType.DMA,
        ],
        compiler_params=pltpu.CompilerParams(use_tc_tiling_on_sc=False),
    )
    def gather(table_hbm, idx_hbm, out_hbm, idx_v, rows_v, sem):
        wid = lax.axis_index("s") * nc + lax.axis_index("c")
        base = wid * b_per_w
        pltpu.sync_copy(idx_hbm.at[pl.ds(wid * n_chunks, n_chunks)], idx_v)
        copies = [
            pltpu.make_async_copy(
                table_hbm.at[idx_v.at[j]],
                rows_v.at[pl.ds(j * chunk, chunk)],
                sem,
            )
            for j in range(n_chunks)
        ]
        for c in copies:
            c.start()
        for c in copies:
            c.wait()
        pltpu.sync_copy(rows_v, out_hbm.at[pl.ds(base, b_per_w)])

    return gather


def kernel(input, embed):
    shape = input.shape
    n = shape[0] * shape[1] * shape[2]
    flatten = input.reshape(-1, _DIM)
    e2 = jnp.sum(embed ** 2, axis=0, keepdims=True)
    ind, dsum = _argmin_call(flatten, embed + embed, e2)
    table = embed.T
    idx2d = ind.reshape(n // 128, 128)
    q = _make_gather(n)(table, idx2d)
    quantize = q.reshape(shape)
    diff = dsum[0, 0] / jnp.float32(n * _DIM)
    quantize = input + lax.stop_gradient(quantize - input)
    embed_ind = ind.reshape(shape[:-1])
    return quantize, diff, embed_ind


# speedup vs baseline: 1.8044x; 1.0185x over previous
import functools

import jax
import jax.numpy as jnp
from jax import lax
from jax.experimental import pallas as pl
from jax.experimental.pallas import tpu as pltpu
from jax.experimental.pallas import tpu_sc as plsc

_DIM = 32
_K = 8192
_BN = 1024


_C = 128


def _chunk_argmin(dot, x2, e2, base, half):
    acc_v = (x2 - dot[:, base:base + _C]) + e2[:, base:base + _C]
    acc_j = jnp.zeros(acc_v.shape, jnp.int32)
    for c in range(1, half // _C):
        sl = slice(base + c * _C, base + (c + 1) * _C)
        dc = (x2 - dot[:, sl]) + e2[:, sl]
        le = acc_v <= dc
        acc_j = jnp.where(le, acc_j, c * _C)
        acc_v = jnp.minimum(acc_v, dc)
    m = jnp.min(acc_v, axis=1)
    lane = jax.lax.broadcasted_iota(jnp.int32, acc_v.shape, 1)
    cand = jnp.where(acc_v == m[:, None], acc_j + lane, _K)
    idx = jnp.min(cand, axis=1)
    return m, idx


def _dist_argmin_body(x_ref, e_ref, e2_ref, ind_ref, dsum_ref):
    x = x_ref[...]
    e2 = e2_ref[...]
    x2 = jnp.sum(x * x, axis=1, keepdims=True)
    dot = jnp.dot(x, e_ref[...])
    half = _K // 2
    m0, i0 = _chunk_argmin(dot, x2, e2, 0, half)
    m1, i1 = _chunk_argmin(dot, x2, e2, half, half)
    m0r = m0.astype(jnp.bfloat16).astype(jnp.float32)
    win1 = m1 < m0r
    ind_ref[...] = jnp.where(win1, i1 + half, i0)
    mind = jnp.where(win1, m1, m0)
    dsum_ref[...] = jnp.broadcast_to(jnp.sum(mind).reshape(1, 1), (8, 128))


def _argmin_call(flatten, embed, e2):
    n = flatten.shape[0]
    grid = (n // _BN,)
    return pl.pallas_call(
        _dist_argmin_body,
        grid=grid,
        in_specs=[
            pl.BlockSpec((_BN, _DIM), lambda i: (i, 0)),
            pl.BlockSpec((_DIM, _K), lambda i: (0, 0)),
            pl.BlockSpec((1, _K), lambda i: (0, 0)),
        ],
        out_specs=[
            pl.BlockSpec((_BN,), lambda i: (i,)),
            pl.BlockSpec((8, 128), lambda i: (i, 0)),
        ],
        out_shape=[
            jax.ShapeDtypeStruct((n,), jnp.int32),
            jax.ShapeDtypeStruct((n // _BN * 8, 128), jnp.float32),
        ],
        compiler_params=pltpu.CompilerParams(
            dimension_semantics=("parallel",),
        ),
    )(flatten, embed, e2)


@functools.lru_cache(maxsize=None)
def _make_gather(n_rows):
    info = plsc.get_sparse_core_info()
    nc, ns = info.num_cores, info.num_subcores
    nw = nc * ns
    b_per_w = n_rows // nw
    chunk = 128
    n_chunks = b_per_w // chunk
    mesh = plsc.VectorSubcoreMesh(core_axis_name="c", subcore_axis_name="s")

    @functools.partial(
        pl.kernel,
        mesh=mesh,
        out_type=jax.ShapeDtypeStruct((n_rows, _DIM), jnp.float32),
        scratch_types=[
            pltpu.VMEM((n_chunks, chunk), jnp.int32),
            pltpu.VMEM((b_per_w, _DIM), jnp.float32),
            pltpu.SemaphoreType.DMA,
        ],
        compiler_params=pltpu.CompilerParams(use_tc_tiling_on_sc=False),
    )
    def gather(table_hbm, idx_hbm, out_hbm, idx_v, rows_v, sem):
        wid = lax.axis_index("s") * nc + lax.axis_index("c")
        base = wid * b_per_w
        pltpu.sync_copy(idx_hbm.at[pl.ds(wid * n_chunks, n_chunks)], idx_v)
        copies = [
            pltpu.make_async_copy(
                table_hbm.at[idx_v.at[j]],
                rows_v.at[pl.ds(j * chunk, chunk)],
                sem,
            )
            for j in range(n_chunks)
        ]
        for c in copies:
            c.start()
        for c in copies:
            c.wait()
        pltpu.sync_copy(rows_v, out_hbm.at[pl.ds(base, b_per_w)])

    return gather


def kernel(input, embed):
    shape = input.shape
    n = shape[0] * shape[1] * shape[2]
    flatten = input.reshape(-1, _DIM)
    e2 = jnp.sum(embed ** 2, axis=0, keepdims=True)
    ind, dsum = _argmin_call(flatten, embed + embed, e2)
    table = embed.T
    idx2d = ind.reshape(n // 128, 128)
    q = _make_gather(n)(table, idx2d)
    quantize = q.reshape(shape)
    diff = jnp.sum(dsum[::8, 0]) / jnp.float32(n * _DIM)
    quantize = input + lax.stop_gradient(quantize - input)
    embed_ind = ind.reshape(shape[:-1])
    return quantize, diff, embed_ind
